# Initial kernel scaffold; baseline (speedup 1.0000x reference)
import jax
import jax.numpy as jnp
from jax import lax
from jax.experimental import pallas as pl
from jax.experimental.pallas import tpu as pltpu

FEAT = 64
NBASIS = 64
NITER = 2
NATOM = 29
NPAD = 32
EDGES = 812
CUTOFF = 5.0
MAXZ = 56

_HI = lax.Precision.HIGHEST


def _mm(a, b):
    return jnp.dot(a, b, precision=_HI)


def _mmT(a, b):
    return lax.dot_general(a, b, (((0,), (0,)), ((), ())), precision=_HI)


def _mpm_body(z_ref, pos_ref, ef_ref, dst_ref, src_ref, logb_ref, kk_ref,
              emb_ref, Wbs_ref, Wbv_ref, Wd1_ref, bd1_ref, Wd1v_ref,
              Wd2_ref, bd2_ref, Wd2v_ref, Wts_ref, Wtv_ref, wout_ref,
              e_ref, f_ref):
    z = z_ref[0]
    posP = pos_ref[0]
    Ef = ef_ref[0]
    dst = dst_ref[...]
    src = src_ref[...]
    logb = logb_ref[...]
    kk = kk_ref[...]
    emb = emb_ref[...]
    wout = wout_ref[...]

    f32 = jnp.float32
    D = (lax.broadcasted_iota(jnp.int32, (NPAD, EDGES), 0) == dst).astype(f32)
    S = (lax.broadcasted_iota(jnp.int32, (NPAD, EDGES), 0) == src).astype(f32)
    ZT = (lax.broadcasted_iota(jnp.int32, (MAXZ, NPAD), 0) == z).astype(f32)
    xs0 = _mmT(ZT, emb)
    amask = (lax.broadcasted_iota(jnp.int32, (NPAD, 1), 0) < NATOM).astype(f32)
    efx = Ef[0:1, 0:1]
    efy = Ef[0:1, 1:2]
    efz = Ef[0:1, 2:3]
    nnk = float(NBASIS - 1) - kk

    def neg_energy(p):
        gp_s = _mmT(S, p)
        gp_d = _mmT(D, p)
        disp = gp_s - gp_d
        r2 = jnp.sum(disp * disp, axis=1, keepdims=True)
        r = jnp.sqrt(r2 + 1e-12)
        unit = disp / r
        u = jnp.clip(r / (r + 1.0), 1e-6, 1.0 - 1e-6)
        lu = jnp.log(u)
        l1mu = jnp.log1p(-u)
        radial = jnp.exp(logb + kk * lu + nnk * l1mu)
        xc = r / CUTOFF
        q = jnp.clip(1.0 - xc * xc, 1e-9, None)
        cut = jnp.where(xc < 1.0, jnp.exp(1.0 - 1.0 / q), 0.0)
        radial = radial * cut
        ux = unit[:, 0:1]
        uy = unit[:, 1:2]
        uz = unit[:, 2:3]

        xs = xs0
        xvx = jnp.zeros((NPAD, FEAT), f32)
        xvy = jnp.zeros((NPAD, FEAT), f32)
        xvz = jnp.zeros((NPAD, FEAT), f32)
        for i in range(NITER):
            rs = _mm(radial, Wbs_ref[i])
            rv = _mm(radial, Wbv_ref[i])
            bvx = ux * rv
            bvy = uy * rv
            bvz = uz * rv
            gs = _mmT(S, xs)
            gvx = _mmT(S, xvx)
            gvy = _mmT(S, xvy)
            gvz = _mmT(S, xvz)
            ms = gs * rs + gvx * bvx + gvy * bvy + gvz * bvz
            ys = _mm(D, ms)
            if i < NITER - 1:
                mvx = gs * bvx + rs * gvx + (gvy * bvz - gvz * bvy)
                mvy = gs * bvy + rs * gvy + (gvz * bvx - gvx * bvz)
                mvz = gs * bvz + rs * gvz + (gvx * bvy - gvy * bvx)
                yvx = _mm(D, mvx)
                yvy = _mm(D, mvy)
                yvz = _mm(D, mvz)
            else:
                yvx = jnp.zeros((NPAD, FEAT), f32)
                yvy = jnp.zeros((NPAD, FEAT), f32)
                yvz = jnp.zeros((NPAD, FEAT), f32)
            xs = xs + ys
            xvx = xvx + yvx
            xvy = xvy + yvy
            xvz = xvz + yvz
            hs = _mm(xs, Wd1_ref[i]) + bd1_ref[i]
            hvx = _mm(xvx, Wd1v_ref[i])
            hvy = _mm(xvy, Wd1v_ref[i])
            hvz = _mm(xvz, Wd1v_ref[i])
            sig = jax.nn.sigmoid(hs)
            hvx = hvx * sig
            hvy = hvy * sig
            hvz = hvz * sig
            hs = hs * sig
            hs = _mm(hs, Wd2_ref[i]) + bd2_ref[i]
            hvx = _mm(hvx, Wd2v_ref[i])
            hvy = _mm(hvy, Wd2v_ref[i])
            hvz = _mm(hvz, Wd2v_ref[i])
            xs = hs + ys
            xvx = hvx + yvx
            xvy = hvy + yvy
            xvz = hvz + yvz
            ts = xs + (xvx * efx + xvy * efy + xvz * efz)
            tvx = xs * efx + xvx + (xvy * efz - xvz * efy)
            tvy = xs * efy + xvy + (xvz * efx - xvx * efz)
            tvz = xs * efz + xvz + (xvx * efy - xvy * efx)
            xs = xs + ts
            xvx = xvx + tvx
            xvy = xvy + tvy
            xvz = xvz + tvz
            us = xs * xs + xvx * xvx + xvy * xvy + xvz * xvz
            uvx = 2.0 * xs * xvx
            uvy = 2.0 * xs * xvy
            uvz = 2.0 * xs * xvz
            xs = _mm(us, Wts_ref[i])
            xvx = _mm(uvx, Wtv_ref[i])
            xvy = _mm(uvy, Wtv_ref[i])
            xvz = _mm(uvz, Wtv_ref[i])
        ae = jnp.sum(xs * wout, axis=1, keepdims=True)
        return -jnp.sum(ae * amask)

    nE, g = jax.value_and_grad(neg_energy)(posP)
    e_ref[0, 0] = -nE
    f_ref[0] = g


def kernel(atomic_numbers, positions, Ef, dst_idx, src_idx, params):
    f32 = jnp.float32
    B, N = atomic_numbers.shape
    z_p = jnp.pad(atomic_numbers.astype(jnp.int32),
                  ((0, 0), (0, NPAD - N))).reshape(B, 1, NPAD)
    pos_p = jnp.pad(positions.astype(f32), ((0, 0), (0, NPAD - N), (0, 0)))
    ef_r = Ef.astype(f32).reshape(B, 1, 3)
    dst_r = dst_idx.astype(jnp.int32).reshape(1, EDGES)
    src_r = src_idx.astype(jnp.int32).reshape(1, EDGES)
    kk = jnp.arange(NBASIS, dtype=f32)
    from jax.scipy.special import gammaln
    nn_ = float(NBASIS - 1)
    logb = (gammaln(nn_ + 1.0) - gammaln(kk + 1.0)
            - gammaln(nn_ - kk + 1.0)).reshape(1, NBASIS)
    kk_r = kk.reshape(1, NBASIS)
    Wb = params['Wb'].astype(f32)
    Wbs = Wb[:, :, 0, :]
    Wbv = Wb[:, :, 1, :]
    bd1 = params['bd1'].astype(f32).reshape(NITER, 1, FEAT)
    bd2 = params['bd2'].astype(f32).reshape(NITER, 1, FEAT)
    wout = params['w_out'].astype(f32).reshape(1, FEAT)
    emb = params['emb'].astype(f32)

    def bcast(shape):
        nd = len(shape)
        return pl.BlockSpec(shape, lambda i: (0,) * nd)

    in_specs = [
        pl.BlockSpec((1, 1, NPAD), lambda i: (i, 0, 0)),
        pl.BlockSpec((1, NPAD, 3), lambda i: (i, 0, 0)),
        pl.BlockSpec((1, 1, 3), lambda i: (i, 0, 0)),
        bcast((1, EDGES)),
        bcast((1, EDGES)),
        bcast((1, NBASIS)),
        bcast((1, NBASIS)),
        bcast((MAXZ, FEAT)),
        bcast((NITER, NBASIS, FEAT)),
        bcast((NITER, NBASIS, FEAT)),
        bcast((NITER, FEAT, FEAT)),
        bcast((NITER, 1, FEAT)),
        bcast((NITER, FEAT, FEAT)),
        bcast((NITER, FEAT, FEAT)),
        bcast((NITER, 1, FEAT)),
        bcast((NITER, FEAT, FEAT)),
        bcast((NITER, FEAT, FEAT)),
        bcast((NITER, FEAT, FEAT)),
        bcast((1, FEAT)),
    ]
    out_specs = [
        pl.BlockSpec((1, 1), lambda i: (i, 0)),
        pl.BlockSpec((1, NPAD, 3), lambda i: (i, 0, 0)),
    ]
    out_shape = [
        jax.ShapeDtypeStruct((B, 1), f32),
        jax.ShapeDtypeStruct((B, NPAD, 3), f32),
    ]
    e_out, f_out = pl.pallas_call(
        _mpm_body,
        grid=(B,),
        in_specs=in_specs,
        out_specs=out_specs,
        out_shape=out_shape,
        compiler_params=pltpu.CompilerParams(
            dimension_semantics=("parallel",)),
    )(z_p, pos_p, ef_r, dst_r, src_r, logb, kk_r, emb, Wbs, Wbv,
      params['Wd1'].astype(f32), bd1, params['Wd1v'].astype(f32),
      params['Wd2'].astype(f32), bd2, params['Wd2v'].astype(f32),
      params['Wts'].astype(f32), params['Wtv'].astype(f32), wout)
    return (e_out[:, 0], f_out[:, :N, :])


# speedup vs baseline: 14.7341x; 14.7341x over previous
import jax
import jax.numpy as jnp
from jax import lax
from jax.experimental import pallas as pl
from jax.experimental.pallas import tpu as pltpu

FEAT = 64
NBASIS = 64
NITER = 2
NATOM = 29
NPAD = 32
EDGES = 812
CUTOFF = 5.0
MAXZ = 56

_HI = lax.Precision.HIGHEST


def _mm(a, b):
    return jnp.dot(a, b, precision=_HI)


def _mmT(a, b):
    return lax.dot_general(a, b, (((0,), (0,)), ((), ())), precision=_HI)


def _mpm_body(z_ref, pos_ref, ef_ref, dst_ref, src_ref, logb_ref, kk_ref,
              emb_ref, Wb2_ref, Wd1_ref, bd1_ref, Wd1v_ref,
              Wd2_ref, bd2_ref, Wd2v_ref, Wts_ref, Wtv_ref, wout_ref,
              e_ref, f_ref):
    z = z_ref[0]
    posP = pos_ref[0]
    ef3 = ef_ref[0]
    dst = dst_ref[...]
    src = src_ref[...]
    logb = logb_ref[...]
    kk = kk_ref[...]
    emb = emb_ref[...]
    wout = wout_ref[...]

    f32 = jnp.float32
    D = (lax.broadcasted_iota(jnp.int32, (NPAD, EDGES), 0) == dst).astype(f32)
    S = (lax.broadcasted_iota(jnp.int32, (NPAD, EDGES), 0) == src).astype(f32)
    ZT = (lax.broadcasted_iota(jnp.int32, (MAXZ, NPAD), 0) == z).astype(f32)
    xs0 = _mmT(ZT, emb)
    amask = (lax.broadcasted_iota(jnp.int32, (NPAD, 1), 0) < NATOM).astype(f32)
    efx = ef3[0:1, :]
    efy = ef3[1:2, :]
    efz = ef3[2:3, :]
    nnk = float(NBASIS - 1) - kk

    SD = S - D

    def neg_energy(p):
        disp = _mmT(SD, p)
        r2 = jnp.sum(disp * disp, axis=1, keepdims=True)
        r = jnp.sqrt(r2 + 1e-12)
        unit = disp / r
        u = jnp.clip(r / (r + 1.0), 1e-6, 1.0 - 1e-6)
        lu = jnp.log(u)
        l1mu = jnp.log(1.0 - u)
        radial = jnp.exp(jnp.maximum(logb + kk * lu + nnk * l1mu, -80.0))
        xc = r / CUTOFF
        q = jnp.clip(1.0 - xc * xc, 1e-9, None)
        cut = jnp.where(xc < 1.0, jnp.exp(jnp.maximum(1.0 - 1.0 / q, -80.0)), 0.0)
        radial = radial * cut
        ux = unit[:, 0:1]
        uy = unit[:, 1:2]
        uz = unit[:, 2:3]

        xs = xs0
        xvx = jnp.zeros((NPAD, FEAT), f32)
        xvy = jnp.zeros((NPAD, FEAT), f32)
        xvz = jnp.zeros((NPAD, FEAT), f32)
        for i in range(NITER):
            rsrv = _mm(radial, Wb2_ref[i])
            rs = rsrv[:, :FEAT]
            rv = rsrv[:, FEAT:]
            bvx = ux * rv
            bvy = uy * rv
            bvz = uz * rv
            Xcat = jnp.concatenate([xs, xvx, xvy, xvz], axis=1)
            G = _mmT(S, Xcat)
            gs = G[:, :FEAT]
            gvx = G[:, FEAT:2 * FEAT]
            gvy = G[:, 2 * FEAT:3 * FEAT]
            gvz = G[:, 3 * FEAT:]
            ms = gs * rs + gvx * bvx + gvy * bvy + gvz * bvz
            if i < NITER - 1:
                mvx = gs * bvx + rs * gvx + (gvy * bvz - gvz * bvy)
                mvy = gs * bvy + rs * gvy + (gvz * bvx - gvx * bvz)
                mvz = gs * bvz + rs * gvz + (gvx * bvy - gvy * bvx)
                Mcat = jnp.concatenate([ms, mvx, mvy, mvz], axis=1)
                Y = _mm(D, Mcat)
                ys = Y[:, :FEAT]
                yvx = Y[:, FEAT:2 * FEAT]
                yvy = Y[:, 2 * FEAT:3 * FEAT]
                yvz = Y[:, 3 * FEAT:]
            else:
                ys = _mm(D, ms)
                yvx = jnp.zeros((NPAD, FEAT), f32)
                yvy = jnp.zeros((NPAD, FEAT), f32)
                yvz = jnp.zeros((NPAD, FEAT), f32)
            xs = xs + ys
            xvx = xvx + yvx
            xvy = xvy + yvy
            xvz = xvz + yvz
            hs = _mm(xs, Wd1_ref[i]) + bd1_ref[i]
            xv3 = jnp.concatenate([xvx, xvy, xvz], axis=0)
            hv3 = _mm(xv3, Wd1v_ref[i])
            sig = jax.nn.sigmoid(hs)
            sig3 = jnp.concatenate([sig, sig, sig], axis=0)
            hv3 = hv3 * sig3
            hs = hs * sig
            hs = _mm(hs, Wd2_ref[i]) + bd2_ref[i]
            hv3 = _mm(hv3, Wd2v_ref[i])
            xs = hs + ys
            xvx = hv3[0:NPAD] + yvx
            xvy = hv3[NPAD:2 * NPAD] + yvy
            xvz = hv3[2 * NPAD:] + yvz
            ts = xs + (xvx * efx + xvy * efy + xvz * efz)
            tvx = xs * efx + xvx + (xvy * efz - xvz * efy)
            tvy = xs * efy + xvy + (xvz * efx - xvx * efz)
            tvz = xs * efz + xvz + (xvx * efy - xvy * efx)
            xs = xs + ts
            xvx = xvx + tvx
            xvy = xvy + tvy
            xvz = xvz + tvz
            us = xs * xs + xvx * xvx + xvy * xvy + xvz * xvz
            uv3 = 2.0 * jnp.concatenate([xs * xvx, xs * xvy, xs * xvz],
                                         axis=0)
            xs = _mm(us, Wts_ref[i])
            xv3n = _mm(uv3, Wtv_ref[i])
            xvx = xv3n[0:NPAD]
            xvy = xv3n[NPAD:2 * NPAD]
            xvz = xv3n[2 * NPAD:]
        ae = jnp.sum(xs * wout, axis=1, keepdims=True)
        return -jnp.sum(ae * amask)

    nE, g = jax.value_and_grad(neg_energy)(posP)
    e_ref[0] = jnp.broadcast_to(-nE, (1, 1))
    f_ref[0] = g


def kernel(atomic_numbers, positions, Ef, dst_idx, src_idx, params):
    f32 = jnp.float32
    B, N = atomic_numbers.shape
    z_p = jnp.pad(atomic_numbers.astype(jnp.int32),
                  ((0, 0), (0, NPAD - N))).reshape(B, 1, NPAD)
    pos_p = jnp.pad(positions.astype(f32), ((0, 0), (0, NPAD - N), (0, 0)))
    ef_r = jnp.broadcast_to(Ef.astype(f32)[:, :, None], (B, 3, FEAT))
    dst_r = dst_idx.astype(jnp.int32).reshape(1, EDGES)
    src_r = src_idx.astype(jnp.int32).reshape(1, EDGES)
    kk = jnp.arange(NBASIS, dtype=f32)
    from jax.scipy.special import gammaln
    nn_ = float(NBASIS - 1)
    logb = (gammaln(nn_ + 1.0) - gammaln(kk + 1.0)
            - gammaln(nn_ - kk + 1.0)).reshape(1, NBASIS)
    kk_r = kk.reshape(1, NBASIS)
    Wb = params['Wb'].astype(f32)
    Wb2 = jnp.concatenate([Wb[:, :, 0, :], Wb[:, :, 1, :]], axis=2)
    bd1 = params['bd1'].astype(f32).reshape(NITER, 1, FEAT)
    bd2 = params['bd2'].astype(f32).reshape(NITER, 1, FEAT)
    wout = params['w_out'].astype(f32).reshape(1, FEAT)
    emb = params['emb'].astype(f32)

    def bcast(shape):
        nd = len(shape)
        return pl.BlockSpec(shape, lambda i: (0,) * nd)

    in_specs = [
        pl.BlockSpec((1, 1, NPAD), lambda i: (i, 0, 0)),
        pl.BlockSpec((1, NPAD, 3), lambda i: (i, 0, 0)),
        pl.BlockSpec((1, 3, FEAT), lambda i: (i, 0, 0)),
        bcast((1, EDGES)),
        bcast((1, EDGES)),
        bcast((1, NBASIS)),
        bcast((1, NBASIS)),
        bcast((MAXZ, FEAT)),
        bcast((NITER, NBASIS, 2 * FEAT)),
        bcast((NITER, FEAT, FEAT)),
        bcast((NITER, 1, FEAT)),
        bcast((NITER, FEAT, FEAT)),
        bcast((NITER, FEAT, FEAT)),
        bcast((NITER, 1, FEAT)),
        bcast((NITER, FEAT, FEAT)),
        bcast((NITER, FEAT, FEAT)),
        bcast((NITER, FEAT, FEAT)),
        bcast((1, FEAT)),
    ]
    out_specs = [
        pl.BlockSpec((1, 1, 1), lambda i: (i, 0, 0)),
        pl.BlockSpec((1, NPAD, 3), lambda i: (i, 0, 0)),
    ]
    out_shape = [
        jax.ShapeDtypeStruct((B, 1, 1), f32),
        jax.ShapeDtypeStruct((B, NPAD, 3), f32),
    ]
    e_out, f_out = pl.pallas_call(
        _mpm_body,
        grid=(B,),
        in_specs=in_specs,
        out_specs=out_specs,
        out_shape=out_shape,
        compiler_params=pltpu.CompilerParams(
            dimension_semantics=("parallel",)),
    )(z_p, pos_p, ef_r, dst_r, src_r, logb, kk_r, emb, Wb2,
      params['Wd1'].astype(f32), bd1, params['Wd1v'].astype(f32),
      params['Wd2'].astype(f32), bd2, params['Wd2v'].astype(f32),
      params['Wts'].astype(f32), params['Wtv'].astype(f32), wout)
    return (e_out[:, 0, 0], f_out[:, :N, :])


# speedup vs baseline: 14.8358x; 1.0069x over previous
import jax
import jax.numpy as jnp
from jax import lax
from jax.experimental import pallas as pl
from jax.experimental.pallas import tpu as pltpu

FEAT = 64
NBASIS = 64
NITER = 2
NATOM = 29
NPAD = 32
EDGES = 812
CUTOFF = 5.0
MAXZ = 56

_HI = lax.Precision.HIGHEST


def _mm(a, b):
    return jnp.dot(a, b, precision=_HI)


def _mmT(a, b):
    return lax.dot_general(a, b, (((0,), (0,)), ((), ())), precision=_HI)


def _mpm_body(z_ref, pos_ref, ef_ref, D_ref, S_ref, SD_ref, logb_ref,
              kk_ref, emb_ref, Wb2_ref, Wd1_ref, bd1_ref, Wd1v_ref,
              Wd2_ref, bd2_ref, Wd2v_ref, Wts_ref, Wtv_ref, wout_ref,
              e_ref, f_ref):
    z = z_ref[0]
    posP = pos_ref[0]
    ef3 = ef_ref[0]
    D = D_ref[...]
    S = S_ref[...]
    SD = SD_ref[...]
    logb = logb_ref[...]
    kk = kk_ref[...]
    emb = emb_ref[...]
    wout = wout_ref[...]

    f32 = jnp.float32
    ZT = (lax.broadcasted_iota(jnp.int32, (MAXZ, NPAD), 0) == z).astype(f32)
    xs0 = _mmT(ZT, emb)
    amask = (lax.broadcasted_iota(jnp.int32, (NPAD, 1), 0) < NATOM).astype(f32)
    efx = ef3[0:1, :]
    efy = ef3[1:2, :]
    efz = ef3[2:3, :]
    nnk = float(NBASIS - 1) - kk

    def neg_energy(p):
        disp = _mmT(SD, p)
        r2 = jnp.sum(disp * disp, axis=1, keepdims=True)
        r = jnp.sqrt(r2 + 1e-12)
        unit = disp / r
        u = jnp.clip(r / (r + 1.0), 1e-6, 1.0 - 1e-6)
        lu = jnp.log(u)
        l1mu = jnp.log(1.0 - u)
        radial = jnp.exp(jnp.maximum(logb + kk * lu + nnk * l1mu, -80.0))
        xc = r / CUTOFF
        q = jnp.clip(1.0 - xc * xc, 1e-9, None)
        cut = jnp.where(xc < 1.0, jnp.exp(jnp.maximum(1.0 - 1.0 / q, -80.0)), 0.0)
        radial = radial * cut
        ux = unit[:, 0:1]
        uy = unit[:, 1:2]
        uz = unit[:, 2:3]

        xs = xs0
        xvx = jnp.zeros((NPAD, FEAT), f32)
        xvy = jnp.zeros((NPAD, FEAT), f32)
        xvz = jnp.zeros((NPAD, FEAT), f32)
        for i in range(NITER):
            rsrv = _mm(radial, Wb2_ref[i])
            rs = rsrv[:, :FEAT]
            rv = rsrv[:, FEAT:]
            bvx = ux * rv
            bvy = uy * rv
            bvz = uz * rv
            Xcat = jnp.concatenate([xs, xvx, xvy, xvz], axis=1)
            G = _mmT(S, Xcat)
            gs = G[:, :FEAT]
            gvx = G[:, FEAT:2 * FEAT]
            gvy = G[:, 2 * FEAT:3 * FEAT]
            gvz = G[:, 3 * FEAT:]
            ms = gs * rs + gvx * bvx + gvy * bvy + gvz * bvz
            if i < NITER - 1:
                mvx = gs * bvx + rs * gvx + (gvy * bvz - gvz * bvy)
                mvy = gs * bvy + rs * gvy + (gvz * bvx - gvx * bvz)
                mvz = gs * bvz + rs * gvz + (gvx * bvy - gvy * bvx)
                Mcat = jnp.concatenate([ms, mvx, mvy, mvz], axis=1)
                Y = _mm(D, Mcat)
                ys = Y[:, :FEAT]
                yvx = Y[:, FEAT:2 * FEAT]
                yvy = Y[:, 2 * FEAT:3 * FEAT]
                yvz = Y[:, 3 * FEAT:]
            else:
                ys = _mm(D, ms)
                yvx = jnp.zeros((NPAD, FEAT), f32)
                yvy = jnp.zeros((NPAD, FEAT), f32)
                yvz = jnp.zeros((NPAD, FEAT), f32)
            xs = xs + ys
            xvx = xvx + yvx
            xvy = xvy + yvy
            xvz = xvz + yvz
            hs = _mm(xs, Wd1_ref[i]) + bd1_ref[i]
            xv3 = jnp.concatenate([xvx, xvy, xvz], axis=0)
            hv3 = _mm(xv3, Wd1v_ref[i])
            sig = jax.nn.sigmoid(hs)
            sig3 = jnp.concatenate([sig, sig, sig], axis=0)
            hv3 = hv3 * sig3
            hs = hs * sig
            hs = _mm(hs, Wd2_ref[i]) + bd2_ref[i]
            hv3 = _mm(hv3, Wd2v_ref[i])
            xs = hs + ys
            xvx = hv3[0:NPAD] + yvx
            xvy = hv3[NPAD:2 * NPAD] + yvy
            xvz = hv3[2 * NPAD:] + yvz
            ts = xs + (xvx * efx + xvy * efy + xvz * efz)
            tvx = xs * efx + xvx + (xvy * efz - xvz * efy)
            tvy = xs * efy + xvy + (xvz * efx - xvx * efz)
            tvz = xs * efz + xvz + (xvx * efy - xvy * efx)
            xs = xs + ts
            xvx = xvx + tvx
            xvy = xvy + tvy
            xvz = xvz + tvz
            us = xs * xs + xvx * xvx + xvy * xvy + xvz * xvz
            uv3 = 2.0 * jnp.concatenate([xs * xvx, xs * xvy, xs * xvz],
                                         axis=0)
            xs = _mm(us, Wts_ref[i])
            xv3n = _mm(uv3, Wtv_ref[i])
            xvx = xv3n[0:NPAD]
            xvy = xv3n[NPAD:2 * NPAD]
            xvz = xv3n[2 * NPAD:]
        ae = jnp.sum(xs * wout, axis=1, keepdims=True)
        return -jnp.sum(ae * amask)

    nE, g = jax.value_and_grad(neg_energy)(posP)
    e_ref[0] = jnp.broadcast_to(-nE, (1, 1))
    f_ref[0] = g


def kernel(atomic_numbers, positions, Ef, dst_idx, src_idx, params):
    f32 = jnp.float32
    B, N = atomic_numbers.shape
    z_p = jnp.pad(atomic_numbers.astype(jnp.int32),
                  ((0, 0), (0, NPAD - N))).reshape(B, 1, NPAD)
    pos_p = jnp.pad(positions.astype(f32), ((0, 0), (0, NPAD - N), (0, 0)))
    ef_r = jnp.broadcast_to(Ef.astype(f32)[:, :, None], (B, 3, FEAT))
    dst_r = dst_idx.astype(jnp.int32).reshape(1, EDGES)
    src_r = src_idx.astype(jnp.int32).reshape(1, EDGES)
    rows = lax.broadcasted_iota(jnp.int32, (NPAD, EDGES), 0)
    D_m = (rows == dst_r).astype(f32)
    S_m = (rows == src_r).astype(f32)
    SD_m = S_m - D_m
    kk = jnp.arange(NBASIS, dtype=f32)
    from jax.scipy.special import gammaln
    nn_ = float(NBASIS - 1)
    logb = (gammaln(nn_ + 1.0) - gammaln(kk + 1.0)
            - gammaln(nn_ - kk + 1.0)).reshape(1, NBASIS)
    kk_r = kk.reshape(1, NBASIS)
    Wb = params['Wb'].astype(f32)
    Wb2 = jnp.concatenate([Wb[:, :, 0, :], Wb[:, :, 1, :]], axis=2)
    bd1 = params['bd1'].astype(f32).reshape(NITER, 1, FEAT)
    bd2 = params['bd2'].astype(f32).reshape(NITER, 1, FEAT)
    wout = params['w_out'].astype(f32).reshape(1, FEAT)
    emb = params['emb'].astype(f32)

    def bcast(shape):
        nd = len(shape)
        return pl.BlockSpec(shape, lambda i: (0,) * nd)

    in_specs = [
        pl.BlockSpec((1, 1, NPAD), lambda i: (i, 0, 0)),
        pl.BlockSpec((1, NPAD, 3), lambda i: (i, 0, 0)),
        pl.BlockSpec((1, 3, FEAT), lambda i: (i, 0, 0)),
        bcast((NPAD, EDGES)),
        bcast((NPAD, EDGES)),
        bcast((NPAD, EDGES)),
        bcast((1, NBASIS)),
        bcast((1, NBASIS)),
        bcast((MAXZ, FEAT)),
        bcast((NITER, NBASIS, 2 * FEAT)),
        bcast((NITER, FEAT, FEAT)),
        bcast((NITER, 1, FEAT)),
        bcast((NITER, FEAT, FEAT)),
        bcast((NITER, FEAT, FEAT)),
        bcast((NITER, 1, FEAT)),
        bcast((NITER, FEAT, FEAT)),
        bcast((NITER, FEAT, FEAT)),
        bcast((NITER, FEAT, FEAT)),
        bcast((1, FEAT)),
    ]
    out_specs = [
        pl.BlockSpec((1, 1, 1), lambda i: (i, 0, 0)),
        pl.BlockSpec((1, NPAD, 3), lambda i: (i, 0, 0)),
    ]
    out_shape = [
        jax.ShapeDtypeStruct((B, 1, 1), f32),
        jax.ShapeDtypeStruct((B, NPAD, 3), f32),
    ]
    e_out, f_out = pl.pallas_call(
        _mpm_body,
        grid=(B,),
        in_specs=in_specs,
        out_specs=out_specs,
        out_shape=out_shape,
        compiler_params=pltpu.CompilerParams(
            dimension_semantics=("parallel",)),
    )(z_p, pos_p, ef_r, D_m, S_m, SD_m, logb, kk_r, emb, Wb2,
      params['Wd1'].astype(f32), bd1, params['Wd1v'].astype(f32),
      params['Wd2'].astype(f32), bd2, params['Wd2v'].astype(f32),
      params['Wts'].astype(f32), params['Wtv'].astype(f32), wout)
    return (e_out[:, 0, 0], f_out[:, :N, :])


# speedup vs baseline: 17.0801x; 1.1513x over previous
import jax
import jax.numpy as jnp
from jax import lax
from jax.experimental import pallas as pl
from jax.experimental.pallas import tpu as pltpu

FEAT = 64
NBASIS = 64
NITER = 2
NATOM = 29
NPAD = 32
EDGES = 812
CUTOFF = 5.0
MAXZ = 56

_HI = lax.Precision.HIGHEST


def _mm(a, b):
    return jnp.dot(a, b, precision=_HI)


def _mmT(a, b):
    return lax.dot_general(a, b, (((0,), (0,)), ((), ())), precision=_HI)


def _mpm_body(z_ref, pos_ref, ef_ref, D_ref, S_ref, SD_ref, logb_ref,
              kk_ref, emb_ref, Wb2_ref, Wd1_ref, bd1_ref, Wd1v_ref,
              Wd2_ref, bd2_ref, Wd2v_ref, Wts_ref, Wtv_ref, wout_ref,
              e_ref, f_ref):
    z = z_ref[0]
    posP = pos_ref[0]
    ef3 = ef_ref[0]
    D = D_ref[...]
    S = S_ref[...]
    SD = SD_ref[...]
    logb = logb_ref[...]
    kk = kk_ref[...]
    emb = emb_ref[...]
    wout = wout_ref[...]

    f32 = jnp.float32
    ZT = (lax.broadcasted_iota(jnp.int32, (MAXZ, NPAD), 0) == z).astype(f32)
    xs0 = _mmT(ZT, emb)
    amask = (lax.broadcasted_iota(jnp.int32, (NPAD, 1), 0) < NATOM).astype(f32)
    efx = ef3[0:1, :]
    efy = ef3[1:2, :]
    efz = ef3[2:3, :]
    nnk = float(NBASIS - 1) - kk

    def neg_energy(p):
        disp = _mmT(SD, p)
        r2 = jnp.sum(disp * disp, axis=1, keepdims=True)
        r = jnp.sqrt(r2 + 1e-12)
        unit = disp / r
        u = jnp.clip(r / (r + 1.0), 1e-6, 1.0 - 1e-6)
        lu = jnp.log(u)
        l1mu = jnp.log(1.0 - u)
        radial = jnp.exp(jnp.maximum(logb + kk * lu + nnk * l1mu, -80.0))
        xc = r / CUTOFF
        q = jnp.clip(1.0 - xc * xc, 1e-9, None)
        cut = jnp.where(xc < 1.0, jnp.exp(jnp.maximum(1.0 - 1.0 / q, -80.0)), 0.0)
        radial = radial * cut
        ux = unit[:, 0:1]
        uy = unit[:, 1:2]
        uz = unit[:, 2:3]

        xs = xs0
        xvx = jnp.zeros((NPAD, FEAT), f32)
        xvy = jnp.zeros((NPAD, FEAT), f32)
        xvz = jnp.zeros((NPAD, FEAT), f32)
        for i in range(NITER):
            rsrv = _mm(radial, Wb2_ref[i])
            rs = rsrv[:, :FEAT]
            rv = rsrv[:, FEAT:]
            bvx = ux * rv
            bvy = uy * rv
            bvz = uz * rv
            gs = _mmT(S, xs)
            gvx = _mmT(S, xvx)
            gvy = _mmT(S, xvy)
            gvz = _mmT(S, xvz)
            ms = gs * rs + gvx * bvx + gvy * bvy + gvz * bvz
            if i < NITER - 1:
                mvx = gs * bvx + rs * gvx + (gvy * bvz - gvz * bvy)
                mvy = gs * bvy + rs * gvy + (gvz * bvx - gvx * bvz)
                mvz = gs * bvz + rs * gvz + (gvx * bvy - gvy * bvx)
                ys = _mm(D, ms)
                yvx = _mm(D, mvx)
                yvy = _mm(D, mvy)
                yvz = _mm(D, mvz)
            else:
                ys = _mm(D, ms)
                yvx = jnp.zeros((NPAD, FEAT), f32)
                yvy = jnp.zeros((NPAD, FEAT), f32)
                yvz = jnp.zeros((NPAD, FEAT), f32)
            xs = xs + ys
            xvx = xvx + yvx
            xvy = xvy + yvy
            xvz = xvz + yvz
            hs = _mm(xs, Wd1_ref[i]) + bd1_ref[i]
            xv3 = jnp.concatenate([xvx, xvy, xvz], axis=0)
            hv3 = _mm(xv3, Wd1v_ref[i])
            sig = jax.nn.sigmoid(hs)
            sig3 = jnp.concatenate([sig, sig, sig], axis=0)
            hv3 = hv3 * sig3
            hs = hs * sig
            hs = _mm(hs, Wd2_ref[i]) + bd2_ref[i]
            hv3 = _mm(hv3, Wd2v_ref[i])
            xs = hs + ys
            xvx = hv3[0:NPAD] + yvx
            xvy = hv3[NPAD:2 * NPAD] + yvy
            xvz = hv3[2 * NPAD:] + yvz
            ts = xs + (xvx * efx + xvy * efy + xvz * efz)
            tvx = xs * efx + xvx + (xvy * efz - xvz * efy)
            tvy = xs * efy + xvy + (xvz * efx - xvx * efz)
            tvz = xs * efz + xvz + (xvx * efy - xvy * efx)
            xs = xs + ts
            xvx = xvx + tvx
            xvy = xvy + tvy
            xvz = xvz + tvz
            us = xs * xs + xvx * xvx + xvy * xvy + xvz * xvz
            uv3 = 2.0 * jnp.concatenate([xs * xvx, xs * xvy, xs * xvz],
                                         axis=0)
            xs = _mm(us, Wts_ref[i])
            xv3n = _mm(uv3, Wtv_ref[i])
            xvx = xv3n[0:NPAD]
            xvy = xv3n[NPAD:2 * NPAD]
            xvz = xv3n[2 * NPAD:]
        ae = jnp.sum(xs * wout, axis=1, keepdims=True)
        return -jnp.sum(ae * amask)

    nE, g = jax.value_and_grad(neg_energy)(posP)
    e_ref[0] = jnp.broadcast_to(-nE, (1, 1))
    f_ref[0] = g


def kernel(atomic_numbers, positions, Ef, dst_idx, src_idx, params):
    f32 = jnp.float32
    B, N = atomic_numbers.shape
    z_p = jnp.pad(atomic_numbers.astype(jnp.int32),
                  ((0, 0), (0, NPAD - N))).reshape(B, 1, NPAD)
    pos_p = jnp.pad(positions.astype(f32), ((0, 0), (0, NPAD - N), (0, 0)))
    ef_r = jnp.broadcast_to(Ef.astype(f32)[:, :, None], (B, 3, FEAT))
    dst_r = dst_idx.astype(jnp.int32).reshape(1, EDGES)
    src_r = src_idx.astype(jnp.int32).reshape(1, EDGES)
    rows = lax.broadcasted_iota(jnp.int32, (NPAD, EDGES), 0)
    D_m = (rows == dst_r).astype(f32)
    S_m = (rows == src_r).astype(f32)
    SD_m = S_m - D_m
    kk = jnp.arange(NBASIS, dtype=f32)
    from jax.scipy.special import gammaln
    nn_ = float(NBASIS - 1)
    logb = (gammaln(nn_ + 1.0) - gammaln(kk + 1.0)
            - gammaln(nn_ - kk + 1.0)).reshape(1, NBASIS)
    kk_r = kk.reshape(1, NBASIS)
    Wb = params['Wb'].astype(f32)
    Wb2 = jnp.concatenate([Wb[:, :, 0, :], Wb[:, :, 1, :]], axis=2)
    bd1 = params['bd1'].astype(f32).reshape(NITER, 1, FEAT)
    bd2 = params['bd2'].astype(f32).reshape(NITER, 1, FEAT)
    wout = params['w_out'].astype(f32).reshape(1, FEAT)
    emb = params['emb'].astype(f32)

    def bcast(shape):
        nd = len(shape)
        return pl.BlockSpec(shape, lambda i: (0,) * nd)

    in_specs = [
        pl.BlockSpec((1, 1, NPAD), lambda i: (i, 0, 0)),
        pl.BlockSpec((1, NPAD, 3), lambda i: (i, 0, 0)),
        pl.BlockSpec((1, 3, FEAT), lambda i: (i, 0, 0)),
        bcast((NPAD, EDGES)),
        bcast((NPAD, EDGES)),
        bcast((NPAD, EDGES)),
        bcast((1, NBASIS)),
        bcast((1, NBASIS)),
        bcast((MAXZ, FEAT)),
        bcast((NITER, NBASIS, 2 * FEAT)),
        bcast((NITER, FEAT, FEAT)),
        bcast((NITER, 1, FEAT)),
        bcast((NITER, FEAT, FEAT)),
        bcast((NITER, FEAT, FEAT)),
        bcast((NITER, 1, FEAT)),
        bcast((NITER, FEAT, FEAT)),
        bcast((NITER, FEAT, FEAT)),
        bcast((NITER, FEAT, FEAT)),
        bcast((1, FEAT)),
    ]
    out_specs = [
        pl.BlockSpec((1, 1, 1), lambda i: (i, 0, 0)),
        pl.BlockSpec((1, NPAD, 3), lambda i: (i, 0, 0)),
    ]
    out_shape = [
        jax.ShapeDtypeStruct((B, 1, 1), f32),
        jax.ShapeDtypeStruct((B, NPAD, 3), f32),
    ]
    e_out, f_out = pl.pallas_call(
        _mpm_body,
        grid=(B,),
        in_specs=in_specs,
        out_specs=out_specs,
        out_shape=out_shape,
        compiler_params=pltpu.CompilerParams(
            dimension_semantics=("parallel",)),
    )(z_p, pos_p, ef_r, D_m, S_m, SD_m, logb, kk_r, emb, Wb2,
      params['Wd1'].astype(f32), bd1, params['Wd1v'].astype(f32),
      params['Wd2'].astype(f32), bd2, params['Wd2v'].astype(f32),
      params['Wts'].astype(f32), params['Wtv'].astype(f32), wout)
    return (e_out[:, 0, 0], f_out[:, :N, :])
